# Initial kernel scaffold; baseline (speedup 1.0000x reference)
import jax
import jax.numpy as jnp
from jax.experimental import pallas as pl


def kernel(x, edge_index, W1_0, b1_0, g_0, be_0, W2_0, b2_0, W1_1, b1_1, g_1, be_1, W2_1, b2_1, W1_2, b1_2, g_2, be_2, W2_2, b2_2):
    raise NotImplementedError("write your pallas kernel here")


# speedup vs baseline: 5.5392x; 5.5392x over previous
import functools
import jax
import jax.numpy as jnp
from jax import lax
from jax.experimental import pallas as pl
from jax.experimental.pallas import tpu as pltpu
from jax.experimental.pallas import tpu_sc as plsc

N = 10000
E = 320000
D = 128

B = 128
NB = E // B
NC = 2
NS = 16
NW = NC * NS
FULL_T = NB // NW
TAIL = NB - FULL_T * NW
NPAD = 10240
RPS = NPAD // NS


def _sc_segment_sum_body(x_hbm, eidx_hbm, zeros_hbm, out_hbm,
                         acc_sh, src_idx, dst_idx, rows, sem):
    c = lax.axis_index("c")
    s = lax.axis_index("s")
    wid = s * NC + c

    pltpu.sync_copy(zeros_hbm, acc_sh.at[pl.ds(s * RPS, RPS)])
    plsc.subcore_barrier()

    def body(t, carry):
        j = t * NW + wid
        pltpu.sync_copy(eidx_hbm.at[0, j], src_idx)
        pltpu.sync_copy(eidx_hbm.at[1, j], dst_idx)
        pltpu.async_copy(x_hbm.at[src_idx], rows, sem).wait()
        pltpu.sync_copy(rows, acc_sh.at[dst_idx], add=True)
        return carry

    lax.fori_loop(0, FULL_T, body, 0)

    @pl.when(wid < TAIL)
    def _tail():
        j = FULL_T * NW + wid
        pltpu.sync_copy(eidx_hbm.at[0, j], src_idx)
        pltpu.sync_copy(eidx_hbm.at[1, j], dst_idx)
        pltpu.async_copy(x_hbm.at[src_idx], rows, sem).wait()
        pltpu.sync_copy(rows, acc_sh.at[dst_idx], add=True)

    plsc.subcore_barrier()
    pltpu.sync_copy(acc_sh.at[pl.ds(s * RPS, RPS)],
                    out_hbm.at[c, pl.ds(s * RPS, RPS)])


@jax.jit
def _sc_segment_sum(x, eidx, zeros_tile):
    mesh = plsc.VectorSubcoreMesh(core_axis_name="c", subcore_axis_name="s")
    f = pl.kernel(
        _sc_segment_sum_body,
        out_type=jax.ShapeDtypeStruct((NC, NPAD, D), jnp.float32),
        mesh=mesh,
        scratch_types=[
            pltpu.VMEM_SHARED((NPAD, D), jnp.float32),
            pltpu.VMEM((B,), jnp.int32),
            pltpu.VMEM((B,), jnp.int32),
            pltpu.VMEM((B, D), jnp.float32),
            pltpu.SemaphoreType.DMA,
        ],
    )
    return f(x, eidx, zeros_tile)


def _tc_dense_body(x_ref, a0_ref, a1_ref, W1_ref, b1_ref, g_ref, be_ref,
                   W2_ref, b2_ref, out_ref):
    h = x_ref[...] + a0_ref[0:N] + a1_ref[0:N]
    h = jnp.dot(h, W1_ref[...], preferred_element_type=jnp.float32) + b1_ref[...]
    mu = jnp.mean(h, axis=0, keepdims=True)
    hc = h - mu
    var = jnp.mean(hc * hc, axis=0, keepdims=True)
    h = hc / jnp.sqrt(var + 1e-5) * g_ref[...] + be_ref[...]
    h = jnp.maximum(h, 0.0)
    out_ref[...] = (
        jnp.dot(h, W2_ref[...], preferred_element_type=jnp.float32) + b2_ref[...]
    )


@jax.jit
def _tc_dense(x, a0, a1, W1, b1, g, be, W2, b2):
    return pl.pallas_call(
        _tc_dense_body,
        out_shape=jax.ShapeDtypeStruct((N, D), jnp.float32),
    )(x, a0, a1, W1, b1.reshape(1, D), g.reshape(1, D), be.reshape(1, D),
      W2, b2.reshape(1, D))


def kernel(x, edge_index,
           W1_0, b1_0, g_0, be_0, W2_0, b2_0,
           W1_1, b1_1, g_1, be_1, W2_1, b2_1,
           W1_2, b1_2, g_2, be_2, W2_2, b2_2):
    eidx = edge_index.reshape(2, NB, B)
    zeros_tile = jnp.zeros((RPS, D), jnp.float32)
    h = x
    for (W1, b1, g, be, W2, b2) in (
        (W1_0, b1_0, g_0, be_0, W2_0, b2_0),
        (W1_1, b1_1, g_1, be_1, W2_1, b2_1),
        (W1_2, b1_2, g_2, be_2, W2_2, b2_2),
    ):
        parts = _sc_segment_sum(h, eidx, zeros_tile)
        h = _tc_dense(h, parts[0], parts[1], W1, b1, g, be, W2, b2)
    return h
